# baseline (speedup 1.0000x reference)
import jax
import jax.numpy as jnp
from jax import lax
from jax.experimental import pallas as pl
from jax.experimental.pallas import tpu as pltpu
from jax.experimental.pallas import tpu_sc as plsc

DIM = 32
LANES = 16
NC, NS = 2, 16
NW = NC * NS
BATCH = 16384
BPW = BATCH // NW
ROWPACK = 128 // DIM
N_ENT = 1000000
NPACK = N_ENT // ROWPACK
NBLK = N_ENT // 128
TAIL = N_ENT - NBLK * 128
TAILROWS = TAIL // ROWPACK
BLK_PER_W = (NBLK + NW - 1) // NW
CHUNK = 128
NCHUNK = BPW // CHUNK
CGROUPS = CHUNK // LANES


def _relayout_body(ut, it, et, tu, ti, te, ou, oi, oe, in_v, out_v, tail_v):
    wid = lax.axis_index("c") * NS + lax.axis_index("s")
    iota = lax.iota(jnp.int32, LANES)
    dlo = iota
    dhi = iota + LANES

    def blk_body(k, carry):
        b = wid + k * NW

        @pl.when(b < NBLK)
        def _():
            col0 = pl.multiple_of(b * 128, 128)
            for tbl, out in ((ut, ou), (it, oi), (et, oe)):
                pltpu.sync_copy(tbl.at[:, pl.ds(col0, 128)], in_v)
                for c in range(128):
                    r, q = c // ROWPACK, c % ROWPACK
                    colv = jnp.full((LANES,), c, jnp.int32)
                    out_v[r, pl.ds(q * DIM, LANES)] = plsc.load_gather(
                        in_v, [dlo, colv])
                    out_v[r, pl.ds(q * DIM + LANES, LANES)] = plsc.load_gather(
                        in_v, [dhi, colv])
                pltpu.sync_copy(out_v, out.at[pl.ds(b * 32, 32), :])
        return carry
    lax.fori_loop(0, BLK_PER_W, blk_body, 0)

    @pl.when(wid == 0)
    def _():
        for tail, out in ((tu, ou), (ti, oi), (te, oe)):
            pltpu.sync_copy(tail, tail_v)
            pltpu.sync_copy(tail_v, out.at[pl.ds(NBLK * 32, TAILROWS), :])


_sc_relayout = pl.kernel(
    _relayout_body,
    out_type=(jax.ShapeDtypeStruct((NPACK, 128), jnp.float32),) * 3,
    mesh=plsc.VectorSubcoreMesh(core_axis_name="c", subcore_axis_name="s"),
    compiler_params=pltpu.CompilerParams(
        needs_layout_passes=False, use_tc_tiling_on_sc=True),
    scratch_types=[
        pltpu.VMEM((DIM, 128), jnp.float32),
        pltpu.VMEM((32, 128), jnp.float32),
        pltpu.VMEM((TAILROWS, 128), jnp.float32),
    ],
)


def _sc_body(uidx_hbm, pidx_hbm, nidx_hbm, user_hbm, item_hbm, ent_hbm,
             out_hbm,
             uidx_v, pidx_v, nidx_v, ublk_v, pblk_v, nblk_v,
             urows, pirows, perows, nirows, nerows, scores_v, sem):
    wid = lax.axis_index("c") * NS + lax.axis_index("s")
    base = wid * BPW

    pltpu.sync_copy(uidx_hbm.at[pl.ds(base, BPW)], uidx_v)
    pltpu.sync_copy(pidx_hbm.at[pl.ds(base, BPW)], pidx_v)
    pltpu.sync_copy(nidx_hbm.at[pl.ds(base, BPW)], nidx_v)

    iota = lax.iota(jnp.int32, LANES)

    def blk_body(g, carry):
        sl = pl.ds(g * LANES, LANES)
        ublk_v[sl] = lax.shift_right_logical(uidx_v[sl], 2)
        pblk_v[sl] = lax.shift_right_logical(pidx_v[sl], 2)
        nblk_v[sl] = lax.shift_right_logical(nidx_v[sl], 2)
        return carry
    lax.fori_loop(0, BPW // LANES, blk_body, 0)

    for ch in range(NCHUNK):
        sl = pl.ds(ch * CHUNK, CHUNK)
        copies = [
            pltpu.async_copy(user_hbm.at[ublk_v.at[sl]], urows, sem),
            pltpu.async_copy(item_hbm.at[pblk_v.at[sl]], pirows, sem),
            pltpu.async_copy(ent_hbm.at[pblk_v.at[sl]], perows, sem),
            pltpu.async_copy(item_hbm.at[nblk_v.at[sl]], nirows, sem),
            pltpu.async_copy(ent_hbm.at[nblk_v.at[sl]], nerows, sem),
        ]
        for c in copies:
            c.wait()

        def dot_body(g, carry, ch=ch):
            rows = g * LANES + iota
            gsl = pl.ds(ch * CHUNK + g * LANES, LANES)
            uoff = (uidx_v[gsl] & 3) * DIM
            poff = (pidx_v[gsl] & 3) * DIM
            noff = (nidx_v[gsl] & 3) * DIM
            acc = jnp.zeros((LANES,), jnp.float32)
            for d in range(DIM):
                uv = plsc.load_gather(urows, [rows, uoff + d])
                pv = (plsc.load_gather(pirows, [rows, poff + d])
                      + plsc.load_gather(perows, [rows, poff + d]))
                nv = (plsc.load_gather(nirows, [rows, noff + d])
                      + plsc.load_gather(nerows, [rows, noff + d]))
                acc = acc + uv * (pv - nv)
            scores_v[gsl] = acc
            return carry
        lax.fori_loop(0, CGROUPS, dot_body, 0)

    pltpu.sync_copy(scores_v, out_hbm.at[pl.ds(base, BPW)])


_sc_diff = pl.kernel(
    _sc_body,
    out_type=jax.ShapeDtypeStruct((BATCH,), jnp.float32),
    mesh=plsc.VectorSubcoreMesh(core_axis_name="c", subcore_axis_name="s"),
    compiler_params=pltpu.CompilerParams(
        needs_layout_passes=False, use_tc_tiling_on_sc=True),
    scratch_types=[
        pltpu.VMEM((BPW,), jnp.int32),
        pltpu.VMEM((BPW,), jnp.int32),
        pltpu.VMEM((BPW,), jnp.int32),
        pltpu.VMEM((BPW,), jnp.int32),
        pltpu.VMEM((BPW,), jnp.int32),
        pltpu.VMEM((BPW,), jnp.int32),
        pltpu.VMEM((CHUNK, 4 * DIM), jnp.float32),
        pltpu.VMEM((CHUNK, 4 * DIM), jnp.float32),
        pltpu.VMEM((CHUNK, 4 * DIM), jnp.float32),
        pltpu.VMEM((CHUNK, 4 * DIM), jnp.float32),
        pltpu.VMEM((CHUNK, 4 * DIM), jnp.float32),
        pltpu.VMEM((BPW,), jnp.float32),
        pltpu.SemaphoreType.DMA,
    ],
)


def _tc_body(x_ref, o_ref):
    x = x_ref[...]
    y = jnp.minimum(x, 0.0) - jnp.log1p(jnp.exp(-jnp.abs(x)))
    o_ref[0, 0] = jnp.sum(y)


_tc_logsig_sum = pl.pallas_call(
    _tc_body,
    out_shape=jax.ShapeDtypeStruct((1, 1), jnp.float32),
    in_specs=[pl.BlockSpec(memory_space=pltpu.VMEM)],
    out_specs=pl.BlockSpec(memory_space=pltpu.SMEM),
)


def kernel(data, name, user_emb_matrix, item_emb_matrix, ent_emb_matrix, Mr_matrix, rel_emb_matrix):
    del name, Mr_matrix, rel_emb_matrix
    tails = [m[NBLK * 128:].reshape(TAILROWS, 128)
             for m in (user_emb_matrix, item_emb_matrix, ent_emb_matrix)]
    t_user, t_item, t_ent = _sc_relayout(
        user_emb_matrix.T, item_emb_matrix.T, ent_emb_matrix.T, *tails)
    diff = _sc_diff(data[:, 0], data[:, 1], data[:, 2], t_user, t_item, t_ent)
    total = _tc_logsig_sum(diff.reshape(BATCH // 128, 128))
    return total[0, 0]


# speedup vs baseline: 1.2155x; 1.2155x over previous
import jax
import jax.numpy as jnp
from jax import lax
from jax.experimental import pallas as pl
from jax.experimental.pallas import tpu as pltpu
from jax.experimental.pallas import tpu_sc as plsc

DIM = 32
LANES = 16
NC, NS = 2, 16
NW = NC * NS
BATCH = 16384
BPW = BATCH // NW
ROWPACK = 128 // DIM
N_ENT = 1000000
NPACK = N_ENT // ROWPACK
CB = 128
ABLK = (N_ENT // CB) // NW * NW
BLK_PER_W = ABLK // NW
PIPE = BLK_PER_W // 2
OROWS = CB // ROWPACK
TAIL = N_ENT - ABLK * CB
TAILROWS = TAIL // ROWPACK
CHUNK = 128
NCHUNK = BPW // CHUNK
CGROUPS = CHUNK // LANES


def _transpose_block(in_v, out_v, dlo, dhi):
    for c in range(CB):
        r, q = c // ROWPACK, c % ROWPACK
        colv = jnp.full((LANES,), c, jnp.int32)
        out_v[r, pl.ds(q * DIM, LANES)] = plsc.load_gather(in_v, [dlo, colv])
        out_v[r, pl.ds(q * DIM + LANES, LANES)] = plsc.load_gather(
            in_v, [dhi, colv])


def _relayout_body(ut, it, et, tu, ti, te, ou, oi, oe,
                   in_v0, in_v1, out_v0, out_v1, tail_v,
                   in_s0, in_s1, out_s0, out_s1):
    wid = lax.axis_index("c") * NS + lax.axis_index("s")
    iota = lax.iota(jnp.int32, LANES)
    dlo = iota
    dhi = iota + LANES
    b0 = wid * BLK_PER_W
    bufs = ((in_v0, in_s0, out_v0, out_s0), (in_v1, in_s1, out_v1, out_s1))

    def _in_slice(tbl, blk):
        return tbl.at[:, pl.ds(pl.multiple_of(blk * CB, CB), CB)]

    for tbl, out in ((ut, ou), (it, oi), (et, oe)):
        pltpu.async_copy(_in_slice(tbl, b0), in_v0, in_s0)
        pltpu.async_copy(_in_slice(tbl, b0 + 1), in_v1, in_s1)

        def body(j, carry, tbl=tbl, out=out):
            for h, (inb, ins, outb, outs) in enumerate(bufs):
                blk = b0 + 2 * j + h

                @pl.when(j > 0)
                def _(outb=outb, outs=outs):
                    pltpu.make_async_copy(
                        outb, out.at[pl.ds(0, OROWS), :], outs).wait()

                pltpu.make_async_copy(_in_slice(tbl, b0), inb, ins).wait()
                _transpose_block(inb, outb, dlo, dhi)
                pltpu.async_copy(
                    outb,
                    out.at[pl.ds(pl.multiple_of(blk * OROWS, 8), OROWS), :],
                    outs)

                @pl.when(2 * j + 2 + h < BLK_PER_W)
                def _(tbl=tbl, blk=blk, inb=inb, ins=ins):
                    pltpu.async_copy(_in_slice(tbl, blk + 2), inb, ins)
            return carry
        lax.fori_loop(0, PIPE, body, 0)

        pltpu.make_async_copy(out_v0, out.at[pl.ds(0, OROWS), :], out_s0).wait()
        pltpu.make_async_copy(out_v1, out.at[pl.ds(0, OROWS), :], out_s1).wait()

    @pl.when(wid == 0)
    def _():
        for tail, out in ((tu, ou), (ti, oi), (te, oe)):
            pltpu.sync_copy(tail, tail_v)
            pltpu.sync_copy(tail_v, out.at[pl.ds(ABLK * OROWS, TAILROWS), :])


_sc_relayout = pl.kernel(
    _relayout_body,
    out_type=(jax.ShapeDtypeStruct((NPACK, 128), jnp.float32),) * 3,
    mesh=plsc.VectorSubcoreMesh(core_axis_name="c", subcore_axis_name="s"),
    compiler_params=pltpu.CompilerParams(
        needs_layout_passes=False, use_tc_tiling_on_sc=True),
    scratch_types=[
        pltpu.VMEM((DIM, CB), jnp.float32),
        pltpu.VMEM((DIM, CB), jnp.float32),
        pltpu.VMEM((OROWS, 128), jnp.float32),
        pltpu.VMEM((OROWS, 128), jnp.float32),
        pltpu.VMEM((TAILROWS, 128), jnp.float32),
        pltpu.SemaphoreType.DMA,
        pltpu.SemaphoreType.DMA,
        pltpu.SemaphoreType.DMA,
        pltpu.SemaphoreType.DMA,
    ],
)


def _sc_body(uidx_hbm, pidx_hbm, nidx_hbm, user_hbm, item_hbm, ent_hbm,
             out_hbm,
             uidx_v, pidx_v, nidx_v, ublk_v, pblk_v, nblk_v,
             urows, pirows, perows, nirows, nerows, scores_v, sem):
    wid = lax.axis_index("c") * NS + lax.axis_index("s")
    base = wid * BPW

    pltpu.sync_copy(uidx_hbm.at[pl.ds(base, BPW)], uidx_v)
    pltpu.sync_copy(pidx_hbm.at[pl.ds(base, BPW)], pidx_v)
    pltpu.sync_copy(nidx_hbm.at[pl.ds(base, BPW)], nidx_v)

    iota = lax.iota(jnp.int32, LANES)

    def blk_body(g, carry):
        sl = pl.ds(g * LANES, LANES)
        ublk_v[sl] = lax.shift_right_logical(uidx_v[sl], 2)
        pblk_v[sl] = lax.shift_right_logical(pidx_v[sl], 2)
        nblk_v[sl] = lax.shift_right_logical(nidx_v[sl], 2)
        return carry
    lax.fori_loop(0, BPW // LANES, blk_body, 0)

    for ch in range(NCHUNK):
        sl = pl.ds(ch * CHUNK, CHUNK)
        copies = [
            pltpu.async_copy(user_hbm.at[ublk_v.at[sl]], urows, sem),
            pltpu.async_copy(item_hbm.at[pblk_v.at[sl]], pirows, sem),
            pltpu.async_copy(ent_hbm.at[pblk_v.at[sl]], perows, sem),
            pltpu.async_copy(item_hbm.at[nblk_v.at[sl]], nirows, sem),
            pltpu.async_copy(ent_hbm.at[nblk_v.at[sl]], nerows, sem),
        ]
        for c in copies:
            c.wait()

        def dot_body(g, carry, ch=ch):
            rows = g * LANES + iota
            gsl = pl.ds(ch * CHUNK + g * LANES, LANES)
            uoff = (uidx_v[gsl] & 3) * DIM
            poff = (pidx_v[gsl] & 3) * DIM
            noff = (nidx_v[gsl] & 3) * DIM
            acc = jnp.zeros((LANES,), jnp.float32)
            for d in range(DIM):
                uv = plsc.load_gather(urows, [rows, uoff + d])
                pv = (plsc.load_gather(pirows, [rows, poff + d])
                      + plsc.load_gather(perows, [rows, poff + d]))
                nv = (plsc.load_gather(nirows, [rows, noff + d])
                      + plsc.load_gather(nerows, [rows, noff + d]))
                acc = acc + uv * (pv - nv)
            scores_v[gsl] = acc
            return carry
        lax.fori_loop(0, CGROUPS, dot_body, 0)

    pltpu.sync_copy(scores_v, out_hbm.at[pl.ds(base, BPW)])


_sc_diff = pl.kernel(
    _sc_body,
    out_type=jax.ShapeDtypeStruct((BATCH,), jnp.float32),
    mesh=plsc.VectorSubcoreMesh(core_axis_name="c", subcore_axis_name="s"),
    compiler_params=pltpu.CompilerParams(
        needs_layout_passes=False, use_tc_tiling_on_sc=True),
    scratch_types=[
        pltpu.VMEM((BPW,), jnp.int32),
        pltpu.VMEM((BPW,), jnp.int32),
        pltpu.VMEM((BPW,), jnp.int32),
        pltpu.VMEM((BPW,), jnp.int32),
        pltpu.VMEM((BPW,), jnp.int32),
        pltpu.VMEM((BPW,), jnp.int32),
        pltpu.VMEM((CHUNK, 4 * DIM), jnp.float32),
        pltpu.VMEM((CHUNK, 4 * DIM), jnp.float32),
        pltpu.VMEM((CHUNK, 4 * DIM), jnp.float32),
        pltpu.VMEM((CHUNK, 4 * DIM), jnp.float32),
        pltpu.VMEM((CHUNK, 4 * DIM), jnp.float32),
        pltpu.VMEM((BPW,), jnp.float32),
        pltpu.SemaphoreType.DMA,
    ],
)


def _tc_body(x_ref, o_ref):
    x = x_ref[...]
    y = jnp.minimum(x, 0.0) - jnp.log1p(jnp.exp(-jnp.abs(x)))
    o_ref[0, 0] = jnp.sum(y)


_tc_logsig_sum = pl.pallas_call(
    _tc_body,
    out_shape=jax.ShapeDtypeStruct((1, 1), jnp.float32),
    in_specs=[pl.BlockSpec(memory_space=pltpu.VMEM)],
    out_specs=pl.BlockSpec(memory_space=pltpu.SMEM),
)


def kernel(data, name, user_emb_matrix, item_emb_matrix, ent_emb_matrix, Mr_matrix, rel_emb_matrix):
    del name, Mr_matrix, rel_emb_matrix
    tails = [m[ABLK * CB:].reshape(TAILROWS, 128)
             for m in (user_emb_matrix, item_emb_matrix, ent_emb_matrix)]
    t_user, t_item, t_ent = _sc_relayout(
        user_emb_matrix.T, item_emb_matrix.T, ent_emb_matrix.T, *tails)
    diff = _sc_diff(data[:, 0], data[:, 1], data[:, 2], t_user, t_item, t_ent)
    total = _tc_logsig_sum(diff.reshape(BATCH // 128, 128))
    return total[0, 0]


# speedup vs baseline: 2.1497x; 1.7686x over previous
import jax
import jax.numpy as jnp
from jax import lax
from jax.experimental import pallas as pl
from jax.experimental.pallas import tpu as pltpu
from jax.experimental.pallas import tpu_sc as plsc

DIM = 32
LANES = 16
NC, NS = 2, 16
NW = NC * NS
BATCH = 16384
BPW = BATCH // NW
ROWPACK = 128 // DIM
N_ENT = 1000000
NPACK = N_ENT // ROWPACK
CB = 128
ABLK = (N_ENT // CB) // NW * NW
BLK_PER_W = ABLK // NW
PIPE = BLK_PER_W // 2
OROWS = CB // ROWPACK
TAIL = N_ENT - ABLK * CB
TAILROWS = TAIL // ROWPACK
CHUNK = 128
NCHUNK = BPW // CHUNK
CGROUPS = CHUNK // LANES


def _transpose_block(in_v, out_v, dlo, dhi):
    for c0 in range(0, CB, 8):
        vals = []
        for c in range(c0, c0 + 8):
            colv = jnp.full((LANES,), c, jnp.int32)
            vals.append((c, plsc.load_gather(in_v, [dlo, colv]),
                         plsc.load_gather(in_v, [dhi, colv])))
        for c, lo, hi in vals:
            r, q = c // ROWPACK, c % ROWPACK
            out_v[r, pl.ds(q * DIM, LANES)] = lo
            out_v[r, pl.ds(q * DIM + LANES, LANES)] = hi


def _relayout_body(ut, it, et, tu, ti, te, ou, oi, oe,
                   in_v0, in_v1, out_v0, out_v1, tail_v,
                   in_s0, in_s1, out_s0, out_s1):
    wid = lax.axis_index("c") * NS + lax.axis_index("s")
    iota = lax.iota(jnp.int32, LANES)
    dlo = iota
    dhi = iota + LANES
    b0 = wid * BLK_PER_W
    bufs = ((in_v0, in_s0, out_v0, out_s0), (in_v1, in_s1, out_v1, out_s1))

    def _in_slice(tbl, blk):
        return tbl.at[:, pl.ds(pl.multiple_of(blk * CB, CB), CB)]

    for tbl, out in ((ut, ou), (it, oi), (et, oe)):
        pltpu.async_copy(_in_slice(tbl, b0), in_v0, in_s0)
        pltpu.async_copy(_in_slice(tbl, b0 + 1), in_v1, in_s1)

        def body(j, carry, tbl=tbl, out=out):
            for h, (inb, ins, outb, outs) in enumerate(bufs):
                blk = b0 + 2 * j + h

                @pl.when(j > 0)
                def _(outb=outb, outs=outs):
                    pltpu.make_async_copy(
                        outb, out.at[pl.ds(0, OROWS), :], outs).wait()

                pltpu.make_async_copy(_in_slice(tbl, b0), inb, ins).wait()
                _transpose_block(inb, outb, dlo, dhi)
                pltpu.async_copy(
                    outb,
                    out.at[pl.ds(pl.multiple_of(blk * OROWS, 8), OROWS), :],
                    outs)

                @pl.when(2 * j + 2 + h < BLK_PER_W)
                def _(tbl=tbl, blk=blk, inb=inb, ins=ins):
                    pltpu.async_copy(_in_slice(tbl, blk + 2), inb, ins)
            return carry
        lax.fori_loop(0, PIPE, body, 0)

        pltpu.make_async_copy(out_v0, out.at[pl.ds(0, OROWS), :], out_s0).wait()
        pltpu.make_async_copy(out_v1, out.at[pl.ds(0, OROWS), :], out_s1).wait()

    @pl.when(wid == 0)
    def _():
        for tail, out in ((tu, ou), (ti, oi), (te, oe)):
            pltpu.sync_copy(tail, tail_v)
            pltpu.sync_copy(tail_v, out.at[pl.ds(ABLK * OROWS, TAILROWS), :])


_sc_relayout = pl.kernel(
    _relayout_body,
    out_type=(jax.ShapeDtypeStruct((NPACK, 128), jnp.float32),) * 3,
    mesh=plsc.VectorSubcoreMesh(core_axis_name="c", subcore_axis_name="s"),
    compiler_params=pltpu.CompilerParams(
        needs_layout_passes=False, use_tc_tiling_on_sc=True),
    scratch_types=[
        pltpu.VMEM((DIM, CB), jnp.float32),
        pltpu.VMEM((DIM, CB), jnp.float32),
        pltpu.VMEM((OROWS, 128), jnp.float32),
        pltpu.VMEM((OROWS, 128), jnp.float32),
        pltpu.VMEM((TAILROWS, 128), jnp.float32),
        pltpu.SemaphoreType.DMA,
        pltpu.SemaphoreType.DMA,
        pltpu.SemaphoreType.DMA,
        pltpu.SemaphoreType.DMA,
    ],
)


def _sc_body(uidx_hbm, pidx_hbm, nidx_hbm, user_hbm, item_hbm, ent_hbm,
             out_hbm,
             uidx_v, pidx_v, nidx_v, ublk_v, pblk_v, nblk_v,
             urows, pirows, perows, nirows, nerows, scores_v, sem):
    wid = lax.axis_index("c") * NS + lax.axis_index("s")
    base = wid * BPW

    pltpu.sync_copy(uidx_hbm.at[pl.ds(base, BPW)], uidx_v)
    pltpu.sync_copy(pidx_hbm.at[pl.ds(base, BPW)], pidx_v)
    pltpu.sync_copy(nidx_hbm.at[pl.ds(base, BPW)], nidx_v)

    iota = lax.iota(jnp.int32, LANES)

    def blk_body(g, carry):
        sl = pl.ds(g * LANES, LANES)
        ublk_v[sl] = lax.shift_right_logical(uidx_v[sl], 2)
        pblk_v[sl] = lax.shift_right_logical(pidx_v[sl], 2)
        nblk_v[sl] = lax.shift_right_logical(nidx_v[sl], 2)
        return carry
    lax.fori_loop(0, BPW // LANES, blk_body, 0)

    for ch in range(NCHUNK):
        sl = pl.ds(ch * CHUNK, CHUNK)
        copies = [
            pltpu.async_copy(user_hbm.at[ublk_v.at[sl]], urows, sem),
            pltpu.async_copy(item_hbm.at[pblk_v.at[sl]], pirows, sem),
            pltpu.async_copy(ent_hbm.at[pblk_v.at[sl]], perows, sem),
            pltpu.async_copy(item_hbm.at[nblk_v.at[sl]], nirows, sem),
            pltpu.async_copy(ent_hbm.at[nblk_v.at[sl]], nerows, sem),
        ]
        for c in copies:
            c.wait()

        def dot_body(g, carry, ch=ch):
            rows = g * LANES + iota
            gsl = pl.ds(ch * CHUNK + g * LANES, LANES)
            uoff = (uidx_v[gsl] & 3) * DIM
            poff = (pidx_v[gsl] & 3) * DIM
            noff = (nidx_v[gsl] & 3) * DIM
            acc = jnp.zeros((LANES,), jnp.float32)
            for d in range(DIM):
                uv = plsc.load_gather(urows, [rows, uoff + d])
                pv = (plsc.load_gather(pirows, [rows, poff + d])
                      + plsc.load_gather(perows, [rows, poff + d]))
                nv = (plsc.load_gather(nirows, [rows, noff + d])
                      + plsc.load_gather(nerows, [rows, noff + d]))
                acc = acc + uv * (pv - nv)
            scores_v[gsl] = acc
            return carry
        lax.fori_loop(0, CGROUPS, dot_body, 0)

    pltpu.sync_copy(scores_v, out_hbm.at[pl.ds(base, BPW)])


_sc_diff = pl.kernel(
    _sc_body,
    out_type=jax.ShapeDtypeStruct((BATCH,), jnp.float32),
    mesh=plsc.VectorSubcoreMesh(core_axis_name="c", subcore_axis_name="s"),
    compiler_params=pltpu.CompilerParams(
        needs_layout_passes=False, use_tc_tiling_on_sc=True),
    scratch_types=[
        pltpu.VMEM((BPW,), jnp.int32),
        pltpu.VMEM((BPW,), jnp.int32),
        pltpu.VMEM((BPW,), jnp.int32),
        pltpu.VMEM((BPW,), jnp.int32),
        pltpu.VMEM((BPW,), jnp.int32),
        pltpu.VMEM((BPW,), jnp.int32),
        pltpu.VMEM((CHUNK, 4 * DIM), jnp.float32),
        pltpu.VMEM((CHUNK, 4 * DIM), jnp.float32),
        pltpu.VMEM((CHUNK, 4 * DIM), jnp.float32),
        pltpu.VMEM((CHUNK, 4 * DIM), jnp.float32),
        pltpu.VMEM((CHUNK, 4 * DIM), jnp.float32),
        pltpu.VMEM((BPW,), jnp.float32),
        pltpu.SemaphoreType.DMA,
    ],
)


def _tc_body(x_ref, o_ref):
    x = x_ref[...]
    y = jnp.minimum(x, 0.0) - jnp.log1p(jnp.exp(-jnp.abs(x)))
    o_ref[0, 0] = jnp.sum(y)


_tc_logsig_sum = pl.pallas_call(
    _tc_body,
    out_shape=jax.ShapeDtypeStruct((1, 1), jnp.float32),
    in_specs=[pl.BlockSpec(memory_space=pltpu.VMEM)],
    out_specs=pl.BlockSpec(memory_space=pltpu.SMEM),
)


def kernel(data, name, user_emb_matrix, item_emb_matrix, ent_emb_matrix, Mr_matrix, rel_emb_matrix):
    del name, Mr_matrix, rel_emb_matrix
    tails = [m[ABLK * CB:].reshape(TAILROWS, 128)
             for m in (user_emb_matrix, item_emb_matrix, ent_emb_matrix)]
    t_user, t_item, t_ent = _sc_relayout(
        user_emb_matrix.T, item_emb_matrix.T, ent_emb_matrix.T, *tails)
    diff = _sc_diff(data[:, 0], data[:, 1], data[:, 2], t_user, t_item, t_ent)
    total = _tc_logsig_sum(diff.reshape(BATCH // 128, 128))
    return total[0, 0]


# speedup vs baseline: 2.1831x; 1.0155x over previous
import jax
import jax.numpy as jnp
from jax import lax
from jax.experimental import pallas as pl
from jax.experimental.pallas import tpu as pltpu
from jax.experimental.pallas import tpu_sc as plsc

DIM = 32
LANES = 16
NC, NS = 2, 16
NW = NC * NS
BATCH = 16384
BPW = BATCH // NW
ROWPACK = 128 // DIM
N_ENT = 1000000
NPACK = N_ENT // ROWPACK
CB = 128
ABLK = (N_ENT // CB) // NW * NW
BLK_PER_W = ABLK // NW
PIPE = BLK_PER_W // 2
OROWS = CB // ROWPACK
TAIL = N_ENT - ABLK * CB
TAILROWS = TAIL // ROWPACK
CHUNK = 128
NCHUNK = BPW // CHUNK
CGROUPS = CHUNK // LANES


NDEEP = 4
PIPE4 = BLK_PER_W // NDEEP


def _transpose_block(in_v, out_v, dlo, dhi):
    def tb(g, carry):
        base = g * 8
        vals = []
        for k in range(8):
            colv = jnp.zeros((LANES,), jnp.int32) + (base + k)
            vals.append((k, plsc.load_gather(in_v, [dlo, colv]),
                         plsc.load_gather(in_v, [dhi, colv])))
        for k, lo, hi in vals:
            r = 2 * g + k // ROWPACK
            q = k % ROWPACK
            out_v[r, pl.ds(q * DIM, LANES)] = lo
            out_v[r, pl.ds(q * DIM + LANES, LANES)] = hi
        return carry
    lax.fori_loop(0, CB // 8, tb, 0)


def _relayout_body(ut, it, et, tu, ti, te, ou, oi, oe,
                   in_v0, in_v1, in_v2, in_v3, out_v0, out_v1, tail_v,
                   in_s0, in_s1, in_s2, in_s3, out_s0, out_s1):
    wid = lax.axis_index("c") * NS + lax.axis_index("s")
    iota = lax.iota(jnp.int32, LANES)
    dlo = iota
    dhi = iota + LANES
    b0 = wid * BLK_PER_W
    ins_v = (in_v0, in_v1, in_v2, in_v3)
    ins_s = (in_s0, in_s1, in_s2, in_s3)
    outs_v = (out_v0, out_v1)
    outs_s = (out_s0, out_s1)

    def _in_slice(tbl, blk):
        return tbl.at[:, pl.ds(pl.multiple_of(blk * CB, CB), CB)]

    for tbl, out in ((ut, ou), (it, oi), (et, oe)):
        for h in range(NDEEP):
            pltpu.async_copy(_in_slice(tbl, b0 + h), ins_v[h], ins_s[h])

        def body(j, carry, tbl=tbl, out=out):
            for h in range(NDEEP):
                inb, ins = ins_v[h], ins_s[h]
                outb, outs = outs_v[h % 2], outs_s[h % 2]
                blk = b0 + NDEEP * j + h

                if h < 2:
                    @pl.when(j > 0)
                    def _(outb=outb, outs=outs, out=out):
                        pltpu.make_async_copy(
                            outb, out.at[pl.ds(0, OROWS), :], outs).wait()
                else:
                    pltpu.make_async_copy(
                        outb, out.at[pl.ds(0, OROWS), :], outs).wait()

                pltpu.make_async_copy(_in_slice(tbl, b0), inb, ins).wait()
                _transpose_block(inb, outb, dlo, dhi)
                pltpu.async_copy(
                    outb,
                    out.at[pl.ds(pl.multiple_of(blk * OROWS, 8), OROWS), :],
                    outs)

                @pl.when(j < PIPE4 - 1)
                def _(tbl=tbl, blk=blk, inb=inb, ins=ins):
                    pltpu.async_copy(_in_slice(tbl, blk + NDEEP), inb, ins)
            return carry
        lax.fori_loop(0, PIPE4, body, 0)

        pltpu.make_async_copy(out_v0, out.at[pl.ds(0, OROWS), :], out_s0).wait()
        pltpu.make_async_copy(out_v1, out.at[pl.ds(0, OROWS), :], out_s1).wait()

    @pl.when(wid == 0)
    def _():
        for tail, out in ((tu, ou), (ti, oi), (te, oe)):
            pltpu.sync_copy(tail, tail_v)
            pltpu.sync_copy(tail_v, out.at[pl.ds(ABLK * OROWS, TAILROWS), :])


_sc_relayout = pl.kernel(
    _relayout_body,
    out_type=(jax.ShapeDtypeStruct((NPACK, 128), jnp.float32),) * 3,
    mesh=plsc.VectorSubcoreMesh(core_axis_name="c", subcore_axis_name="s"),
    compiler_params=pltpu.CompilerParams(
        needs_layout_passes=False, use_tc_tiling_on_sc=True),
    scratch_types=[
        pltpu.VMEM((DIM, CB), jnp.float32),
        pltpu.VMEM((DIM, CB), jnp.float32),
        pltpu.VMEM((DIM, CB), jnp.float32),
        pltpu.VMEM((DIM, CB), jnp.float32),
        pltpu.VMEM((OROWS, 128), jnp.float32),
        pltpu.VMEM((OROWS, 128), jnp.float32),
        pltpu.VMEM((TAILROWS, 128), jnp.float32),
        pltpu.SemaphoreType.DMA,
        pltpu.SemaphoreType.DMA,
        pltpu.SemaphoreType.DMA,
        pltpu.SemaphoreType.DMA,
        pltpu.SemaphoreType.DMA,
        pltpu.SemaphoreType.DMA,
    ],
)


def _sc_body(uidx_hbm, pidx_hbm, nidx_hbm, user_hbm, item_hbm, ent_hbm,
             out_hbm,
             uidx_v, pidx_v, nidx_v, ublk_v, pblk_v, nblk_v,
             urows, pirows, perows, nirows, nerows, scores_v, sem):
    wid = lax.axis_index("c") * NS + lax.axis_index("s")
    base = wid * BPW

    pltpu.sync_copy(uidx_hbm.at[pl.ds(base, BPW)], uidx_v)
    pltpu.sync_copy(pidx_hbm.at[pl.ds(base, BPW)], pidx_v)
    pltpu.sync_copy(nidx_hbm.at[pl.ds(base, BPW)], nidx_v)

    iota = lax.iota(jnp.int32, LANES)

    def blk_body(g, carry):
        sl = pl.ds(g * LANES, LANES)
        ublk_v[sl] = lax.shift_right_logical(uidx_v[sl], 2)
        pblk_v[sl] = lax.shift_right_logical(pidx_v[sl], 2)
        nblk_v[sl] = lax.shift_right_logical(nidx_v[sl], 2)
        return carry
    lax.fori_loop(0, BPW // LANES, blk_body, 0)

    for ch in range(NCHUNK):
        sl = pl.ds(ch * CHUNK, CHUNK)
        copies = [
            pltpu.async_copy(user_hbm.at[ublk_v.at[sl]], urows, sem),
            pltpu.async_copy(item_hbm.at[pblk_v.at[sl]], pirows, sem),
            pltpu.async_copy(ent_hbm.at[pblk_v.at[sl]], perows, sem),
            pltpu.async_copy(item_hbm.at[nblk_v.at[sl]], nirows, sem),
            pltpu.async_copy(ent_hbm.at[nblk_v.at[sl]], nerows, sem),
        ]
        for c in copies:
            c.wait()

        def dot_body(g, carry, ch=ch):
            rows = g * LANES + iota
            gsl = pl.ds(ch * CHUNK + g * LANES, LANES)
            uoff = (uidx_v[gsl] & 3) * DIM
            poff = (pidx_v[gsl] & 3) * DIM
            noff = (nidx_v[gsl] & 3) * DIM
            acc = jnp.zeros((LANES,), jnp.float32)
            for d in range(DIM):
                uv = plsc.load_gather(urows, [rows, uoff + d])
                pv = (plsc.load_gather(pirows, [rows, poff + d])
                      + plsc.load_gather(perows, [rows, poff + d]))
                nv = (plsc.load_gather(nirows, [rows, noff + d])
                      + plsc.load_gather(nerows, [rows, noff + d]))
                acc = acc + uv * (pv - nv)
            scores_v[gsl] = acc
            return carry
        lax.fori_loop(0, CGROUPS, dot_body, 0)

    pltpu.sync_copy(scores_v, out_hbm.at[pl.ds(base, BPW)])


_sc_diff = pl.kernel(
    _sc_body,
    out_type=jax.ShapeDtypeStruct((BATCH,), jnp.float32),
    mesh=plsc.VectorSubcoreMesh(core_axis_name="c", subcore_axis_name="s"),
    compiler_params=pltpu.CompilerParams(
        needs_layout_passes=False, use_tc_tiling_on_sc=True),
    scratch_types=[
        pltpu.VMEM((BPW,), jnp.int32),
        pltpu.VMEM((BPW,), jnp.int32),
        pltpu.VMEM((BPW,), jnp.int32),
        pltpu.VMEM((BPW,), jnp.int32),
        pltpu.VMEM((BPW,), jnp.int32),
        pltpu.VMEM((BPW,), jnp.int32),
        pltpu.VMEM((CHUNK, 4 * DIM), jnp.float32),
        pltpu.VMEM((CHUNK, 4 * DIM), jnp.float32),
        pltpu.VMEM((CHUNK, 4 * DIM), jnp.float32),
        pltpu.VMEM((CHUNK, 4 * DIM), jnp.float32),
        pltpu.VMEM((CHUNK, 4 * DIM), jnp.float32),
        pltpu.VMEM((BPW,), jnp.float32),
        pltpu.SemaphoreType.DMA,
    ],
)


def _tc_body(x_ref, o_ref):
    x = x_ref[...]
    y = jnp.minimum(x, 0.0) - jnp.log1p(jnp.exp(-jnp.abs(x)))
    o_ref[0, 0] = jnp.sum(y)


_tc_logsig_sum = pl.pallas_call(
    _tc_body,
    out_shape=jax.ShapeDtypeStruct((1, 1), jnp.float32),
    in_specs=[pl.BlockSpec(memory_space=pltpu.VMEM)],
    out_specs=pl.BlockSpec(memory_space=pltpu.SMEM),
)


def kernel(data, name, user_emb_matrix, item_emb_matrix, ent_emb_matrix, Mr_matrix, rel_emb_matrix):
    del name, Mr_matrix, rel_emb_matrix
    tails = [m[ABLK * CB:].reshape(TAILROWS, 128)
             for m in (user_emb_matrix, item_emb_matrix, ent_emb_matrix)]
    t_user, t_item, t_ent = _sc_relayout(
        user_emb_matrix.T, item_emb_matrix.T, ent_emb_matrix.T, *tails)
    diff = _sc_diff(data[:, 0], data[:, 1], data[:, 2], t_user, t_item, t_ent)
    total = _tc_logsig_sum(diff.reshape(BATCH // 128, 128))
    return total[0, 0]


# speedup vs baseline: 2.1867x; 1.0016x over previous
import jax
import jax.numpy as jnp
from jax import lax
from jax.experimental import pallas as pl
from jax.experimental.pallas import tpu as pltpu
from jax.experimental.pallas import tpu_sc as plsc

DIM = 32
LANES = 16
NC, NS = 2, 16
NW = NC * NS
BATCH = 16384
BPW = BATCH // NW
ROWPACK = 128 // DIM
N_ENT = 1000000
NPACK = N_ENT // ROWPACK
CB = 128
ABLK = (N_ENT // CB) // NW * NW
BLK_PER_W = ABLK // NW
PIPE = BLK_PER_W // 2
OROWS = CB // ROWPACK
TAIL = N_ENT - ABLK * CB
TAILROWS = TAIL // ROWPACK
CHUNK = 128
NCHUNK = BPW // CHUNK
CGROUPS = CHUNK // LANES


NDEEP = 4
PIPE4 = BLK_PER_W // NDEEP


def _transpose_block(in_v, out_v, dlo, dhi):
    def tb(g, carry):
        base = g * 8
        vals = []
        for k in range(8):
            colv = jnp.zeros((LANES,), jnp.int32) + (base + k)
            vals.append((k, plsc.load_gather(in_v, [dlo, colv]),
                         plsc.load_gather(in_v, [dhi, colv])))
        for k, lo, hi in vals:
            r = 2 * g + k // ROWPACK
            q = k % ROWPACK
            out_v[r, pl.ds(q * DIM, LANES)] = lo
            out_v[r, pl.ds(q * DIM + LANES, LANES)] = hi
        return carry
    lax.fori_loop(0, CB // 8, tb, 0)


def _relayout_body(ut, it, et, tu, ti, te, ou, oi, oe,
                   in_v0, in_v1, in_v2, in_v3, out_v0, out_v1, tail_v,
                   in_s0, in_s1, in_s2, in_s3, out_s0, out_s1):
    wid = lax.axis_index("c") * NS + lax.axis_index("s")
    iota = lax.iota(jnp.int32, LANES)
    dlo = iota
    dhi = iota + LANES
    b0 = wid * BLK_PER_W
    ins_v = (in_v0, in_v1, in_v2, in_v3)
    ins_s = (in_s0, in_s1, in_s2, in_s3)
    outs_v = (out_v0, out_v1)
    outs_s = (out_s0, out_s1)

    def _in_slice(tbl, blk):
        return tbl.at[:, pl.ds(pl.multiple_of(blk * CB, CB), CB)]

    def _in_dst(inb):
        return inb.at[:, pl.ds(0, CB)]

    for tbl, out in ((ut, ou), (it, oi), (et, oe)):
        for h in range(NDEEP):
            pltpu.async_copy(_in_slice(tbl, b0 + h), _in_dst(ins_v[h]), ins_s[h])

        def body(j, carry, tbl=tbl, out=out):
            for h in range(NDEEP):
                inb, ins = ins_v[h], ins_s[h]
                outb, outs = outs_v[h % 2], outs_s[h % 2]
                blk = b0 + NDEEP * j + h

                if h < 2:
                    @pl.when(j > 0)
                    def _(outb=outb, outs=outs, out=out):
                        pltpu.make_async_copy(
                            outb, out.at[pl.ds(0, OROWS), :], outs).wait()
                else:
                    pltpu.make_async_copy(
                        outb, out.at[pl.ds(0, OROWS), :], outs).wait()

                pltpu.make_async_copy(_in_slice(tbl, b0), _in_dst(inb), ins).wait()
                _transpose_block(inb, outb, dlo, dhi)
                pltpu.async_copy(
                    outb,
                    out.at[pl.ds(pl.multiple_of(blk * OROWS, 8), OROWS), :],
                    outs)

                @pl.when(j < PIPE4 - 1)
                def _(tbl=tbl, blk=blk, inb=inb, ins=ins):
                    pltpu.async_copy(_in_slice(tbl, blk + NDEEP), _in_dst(inb), ins)
            return carry
        lax.fori_loop(0, PIPE4, body, 0)

        pltpu.make_async_copy(out_v0, out.at[pl.ds(0, OROWS), :], out_s0).wait()
        pltpu.make_async_copy(out_v1, out.at[pl.ds(0, OROWS), :], out_s1).wait()

    @pl.when(wid == 0)
    def _():
        for tail, out in ((tu, ou), (ti, oi), (te, oe)):
            pltpu.sync_copy(tail, tail_v)
            pltpu.sync_copy(tail_v, out.at[pl.ds(ABLK * OROWS, TAILROWS), :])


_sc_relayout = pl.kernel(
    _relayout_body,
    out_type=(jax.ShapeDtypeStruct((NPACK, 128), jnp.float32),) * 3,
    mesh=plsc.VectorSubcoreMesh(core_axis_name="c", subcore_axis_name="s"),
    compiler_params=pltpu.CompilerParams(
        needs_layout_passes=False, use_tc_tiling_on_sc=True),
    scratch_types=[
        pltpu.VMEM((DIM, CB + 1), jnp.float32),
        pltpu.VMEM((DIM, CB + 1), jnp.float32),
        pltpu.VMEM((DIM, CB + 1), jnp.float32),
        pltpu.VMEM((DIM, CB + 1), jnp.float32),
        pltpu.VMEM((OROWS, 128), jnp.float32),
        pltpu.VMEM((OROWS, 128), jnp.float32),
        pltpu.VMEM((TAILROWS, 128), jnp.float32),
        pltpu.SemaphoreType.DMA,
        pltpu.SemaphoreType.DMA,
        pltpu.SemaphoreType.DMA,
        pltpu.SemaphoreType.DMA,
        pltpu.SemaphoreType.DMA,
        pltpu.SemaphoreType.DMA,
    ],
)


def _sc_body(uidx_hbm, pidx_hbm, nidx_hbm, user_hbm, item_hbm, ent_hbm,
             out_hbm,
             uidx_v, pidx_v, nidx_v, ublk_v, pblk_v, nblk_v,
             urows, pirows, perows, nirows, nerows, scores_v, sem):
    wid = lax.axis_index("c") * NS + lax.axis_index("s")
    base = wid * BPW

    pltpu.sync_copy(uidx_hbm.at[pl.ds(base, BPW)], uidx_v)
    pltpu.sync_copy(pidx_hbm.at[pl.ds(base, BPW)], pidx_v)
    pltpu.sync_copy(nidx_hbm.at[pl.ds(base, BPW)], nidx_v)

    iota = lax.iota(jnp.int32, LANES)

    def blk_body(g, carry):
        sl = pl.ds(g * LANES, LANES)
        ublk_v[sl] = lax.shift_right_logical(uidx_v[sl], 2)
        pblk_v[sl] = lax.shift_right_logical(pidx_v[sl], 2)
        nblk_v[sl] = lax.shift_right_logical(nidx_v[sl], 2)
        return carry
    lax.fori_loop(0, BPW // LANES, blk_body, 0)

    for ch in range(NCHUNK):
        sl = pl.ds(ch * CHUNK, CHUNK)
        copies = [
            pltpu.async_copy(user_hbm.at[ublk_v.at[sl]], urows, sem),
            pltpu.async_copy(item_hbm.at[pblk_v.at[sl]], pirows, sem),
            pltpu.async_copy(ent_hbm.at[pblk_v.at[sl]], perows, sem),
            pltpu.async_copy(item_hbm.at[nblk_v.at[sl]], nirows, sem),
            pltpu.async_copy(ent_hbm.at[nblk_v.at[sl]], nerows, sem),
        ]
        for c in copies:
            c.wait()

        def dot_body(g, carry, ch=ch):
            rows = g * LANES + iota
            gsl = pl.ds(ch * CHUNK + g * LANES, LANES)
            uoff = (uidx_v[gsl] & 3) * DIM
            poff = (pidx_v[gsl] & 3) * DIM
            noff = (nidx_v[gsl] & 3) * DIM
            acc = jnp.zeros((LANES,), jnp.float32)
            for d in range(DIM):
                uv = plsc.load_gather(urows, [rows, uoff + d])
                pv = (plsc.load_gather(pirows, [rows, poff + d])
                      + plsc.load_gather(perows, [rows, poff + d]))
                nv = (plsc.load_gather(nirows, [rows, noff + d])
                      + plsc.load_gather(nerows, [rows, noff + d]))
                acc = acc + uv * (pv - nv)
            scores_v[gsl] = acc
            return carry
        lax.fori_loop(0, CGROUPS, dot_body, 0)

    pltpu.sync_copy(scores_v, out_hbm.at[pl.ds(base, BPW)])


_sc_diff = pl.kernel(
    _sc_body,
    out_type=jax.ShapeDtypeStruct((BATCH,), jnp.float32),
    mesh=plsc.VectorSubcoreMesh(core_axis_name="c", subcore_axis_name="s"),
    compiler_params=pltpu.CompilerParams(
        needs_layout_passes=False, use_tc_tiling_on_sc=True),
    scratch_types=[
        pltpu.VMEM((BPW,), jnp.int32),
        pltpu.VMEM((BPW,), jnp.int32),
        pltpu.VMEM((BPW,), jnp.int32),
        pltpu.VMEM((BPW,), jnp.int32),
        pltpu.VMEM((BPW,), jnp.int32),
        pltpu.VMEM((BPW,), jnp.int32),
        pltpu.VMEM((CHUNK, 4 * DIM), jnp.float32),
        pltpu.VMEM((CHUNK, 4 * DIM), jnp.float32),
        pltpu.VMEM((CHUNK, 4 * DIM), jnp.float32),
        pltpu.VMEM((CHUNK, 4 * DIM), jnp.float32),
        pltpu.VMEM((CHUNK, 4 * DIM), jnp.float32),
        pltpu.VMEM((BPW,), jnp.float32),
        pltpu.SemaphoreType.DMA,
    ],
)


def _tc_body(x_ref, o_ref):
    x = x_ref[...]
    y = jnp.minimum(x, 0.0) - jnp.log1p(jnp.exp(-jnp.abs(x)))
    o_ref[0, 0] = jnp.sum(y)


_tc_logsig_sum = pl.pallas_call(
    _tc_body,
    out_shape=jax.ShapeDtypeStruct((1, 1), jnp.float32),
    in_specs=[pl.BlockSpec(memory_space=pltpu.VMEM)],
    out_specs=pl.BlockSpec(memory_space=pltpu.SMEM),
)


def kernel(data, name, user_emb_matrix, item_emb_matrix, ent_emb_matrix, Mr_matrix, rel_emb_matrix):
    del name, Mr_matrix, rel_emb_matrix
    tails = [m[ABLK * CB:].reshape(TAILROWS, 128)
             for m in (user_emb_matrix, item_emb_matrix, ent_emb_matrix)]
    t_user, t_item, t_ent = _sc_relayout(
        user_emb_matrix.T, item_emb_matrix.T, ent_emb_matrix.T, *tails)
    diff = _sc_diff(data[:, 0], data[:, 1], data[:, 2], t_user, t_item, t_ent)
    total = _tc_logsig_sum(diff.reshape(BATCH // 128, 128))
    return total[0, 0]


# speedup vs baseline: 2.6732x; 1.2225x over previous
import jax
import jax.numpy as jnp
from jax import lax
from jax.experimental import pallas as pl
from jax.experimental.pallas import tpu as pltpu
from jax.experimental.pallas import tpu_sc as plsc

DIM = 32
LANES = 16
NC, NS = 2, 16
NW = NC * NS
BATCH = 16384
BPW = BATCH // NW
ROWPACK = 128 // DIM
CHUNK = 128
NCHUNK = BPW // CHUNK
CGROUPS = CHUNK // LANES


def _sc_body(uidx_hbm, pidx_hbm, nidx_hbm, user_hbm, item_hbm, ent_hbm,
             out_hbm,
             uidx_v, pidx_v, nidx_v, ublk_v, pblk_v, nblk_v,
             urows, pirows, perows, nirows, nerows, scores_v, sem):
    wid = lax.axis_index("c") * NS + lax.axis_index("s")
    base = wid * BPW

    pltpu.sync_copy(uidx_hbm.at[pl.ds(base, BPW)], uidx_v)
    pltpu.sync_copy(pidx_hbm.at[pl.ds(base, BPW)], pidx_v)
    pltpu.sync_copy(nidx_hbm.at[pl.ds(base, BPW)], nidx_v)

    iota = lax.iota(jnp.int32, LANES)

    def blk_body(g, carry):
        sl = pl.ds(g * LANES, LANES)
        ublk_v[sl] = lax.shift_right_logical(uidx_v[sl], 2)
        pblk_v[sl] = lax.shift_right_logical(pidx_v[sl], 2)
        nblk_v[sl] = lax.shift_right_logical(nidx_v[sl], 2)
        return carry
    lax.fori_loop(0, BPW // LANES, blk_body, 0)

    for ch in range(NCHUNK):
        sl = pl.ds(ch * CHUNK, CHUNK)
        copies = [
            pltpu.async_copy(user_hbm.at[ublk_v.at[sl]], urows, sem),
            pltpu.async_copy(item_hbm.at[pblk_v.at[sl]], pirows, sem),
            pltpu.async_copy(ent_hbm.at[pblk_v.at[sl]], perows, sem),
            pltpu.async_copy(item_hbm.at[nblk_v.at[sl]], nirows, sem),
            pltpu.async_copy(ent_hbm.at[nblk_v.at[sl]], nerows, sem),
        ]
        for c in copies:
            c.wait()

        def dot_body(g, carry, ch=ch):
            rows = g * LANES + iota
            gsl = pl.ds(ch * CHUNK + g * LANES, LANES)
            uoff = (uidx_v[gsl] & 3) * DIM
            poff = (pidx_v[gsl] & 3) * DIM
            noff = (nidx_v[gsl] & 3) * DIM
            acc = jnp.zeros((LANES,), jnp.float32)
            for s in range(LANES):
                for h in range(DIM // LANES):
                    dv = ((iota + s) & (LANES - 1)) + h * LANES
                    uv = plsc.load_gather(urows, [rows, uoff + dv])
                    pv = (plsc.load_gather(pirows, [rows, poff + dv])
                          + plsc.load_gather(perows, [rows, poff + dv]))
                    nv = (plsc.load_gather(nirows, [rows, noff + dv])
                          + plsc.load_gather(nerows, [rows, noff + dv]))
                    acc = acc + uv * (pv - nv)
            scores_v[gsl] = acc
            return carry
        lax.fori_loop(0, CGROUPS, dot_body, 0)

    pltpu.sync_copy(scores_v, out_hbm.at[pl.ds(base, BPW)])


_sc_diff = pl.kernel(
    _sc_body,
    out_type=jax.ShapeDtypeStruct((BATCH,), jnp.float32),
    mesh=plsc.VectorSubcoreMesh(core_axis_name="c", subcore_axis_name="s"),
    compiler_params=pltpu.CompilerParams(
        needs_layout_passes=False, use_tc_tiling_on_sc=True),
    scratch_types=[
        pltpu.VMEM((BPW,), jnp.int32),
        pltpu.VMEM((BPW,), jnp.int32),
        pltpu.VMEM((BPW,), jnp.int32),
        pltpu.VMEM((BPW,), jnp.int32),
        pltpu.VMEM((BPW,), jnp.int32),
        pltpu.VMEM((BPW,), jnp.int32),
        pltpu.VMEM((CHUNK, 4 * DIM), jnp.float32),
        pltpu.VMEM((CHUNK, 4 * DIM), jnp.float32),
        pltpu.VMEM((CHUNK, 4 * DIM), jnp.float32),
        pltpu.VMEM((CHUNK, 4 * DIM), jnp.float32),
        pltpu.VMEM((CHUNK, 4 * DIM), jnp.float32),
        pltpu.VMEM((BPW,), jnp.float32),
        pltpu.SemaphoreType.DMA,
    ],
)


def _tc_body(x_ref, o_ref):
    x = x_ref[...]
    y = jnp.minimum(x, 0.0) - jnp.log1p(jnp.exp(-jnp.abs(x)))
    o_ref[0, 0] = jnp.sum(y)


_tc_logsig_sum = pl.pallas_call(
    _tc_body,
    out_shape=jax.ShapeDtypeStruct((1, 1), jnp.float32),
    in_specs=[pl.BlockSpec(memory_space=pltpu.VMEM)],
    out_specs=pl.BlockSpec(memory_space=pltpu.SMEM),
)


def kernel(data, name, user_emb_matrix, item_emb_matrix, ent_emb_matrix, Mr_matrix, rel_emb_matrix):
    del name, Mr_matrix, rel_emb_matrix
    n_packed = user_emb_matrix.shape[0] // ROWPACK
    t_user = user_emb_matrix.reshape(n_packed, ROWPACK * DIM)
    t_item = item_emb_matrix.reshape(n_packed, ROWPACK * DIM)
    t_ent = ent_emb_matrix.reshape(n_packed, ROWPACK * DIM)
    diff = _sc_diff(data[:, 0], data[:, 1], data[:, 2], t_user, t_item, t_ent)
    total = _tc_logsig_sum(diff.reshape(BATCH // 128, 128))
    return total[0, 0]
